# Initial kernel scaffold; baseline (speedup 1.0000x reference)
import jax
import jax.numpy as jnp
from jax.experimental import pallas as pl


def kernel(queries, keys, values, wq, wk, wv, wo):
    raise NotImplementedError("write your pallas kernel here")


# speedup vs baseline: 4.2311x; 4.2311x over previous
import functools
import math

import numpy as np
import jax
import jax.numpy as jnp
from jax import lax
from jax.experimental import pallas as pl

_B = 2
_L = 2048
_DM = 1024
_H = 16
_DH = 64
_K = int(math.floor(math.log(_L)))
_NFP = 1152

_PREC = lax.Precision.HIGHEST
_PREC_PROJ = lax.Precision.DEFAULT


def _dft_consts():
    l = np.arange(_L, dtype=np.float64)
    k = np.arange(_NFP, dtype=np.float64)
    ang = 2.0 * np.pi * np.outer(k, l) / _L
    keep = (np.arange(_NFP) <= _L // 2).astype(np.float64)[:, None]
    fct = np.cos(ang) * keep
    fst = -np.sin(ang) * keep
    ck = np.where((np.arange(_NFP) == 0) | (np.arange(_NFP) == _L // 2), 1.0, 2.0)
    ck = ck * keep[:, 0] / _L
    act = (np.cos(ang) * ck[:, None]).T
    ast = (-np.sin(ang) * ck[:, None]).T
    return (jnp.asarray(fct, jnp.float32), jnp.asarray(fst, jnp.float32),
            jnp.asarray(act, jnp.float32), jnp.asarray(ast, jnp.float32))


def _mm_kernel(x_ref, w_ref, o_ref):
    o_ref[...] = jnp.dot(x_ref[...], w_ref[...],
                         preferred_element_type=jnp.float32,
                         precision=_PREC_PROJ)


def _matmul(x, w, bm=512):
    m, kk = x.shape
    n = w.shape[1]
    return pl.pallas_call(
        _mm_kernel,
        grid=(m // bm,),
        in_specs=[pl.BlockSpec((bm, kk), lambda i: (i, 0)),
                  pl.BlockSpec((kk, n), lambda i: (0, 0))],
        out_specs=pl.BlockSpec((bm, n), lambda i: (i, 0)),
        out_shape=jax.ShapeDtypeStruct((m, n), jnp.float32),
    )(x, w)


def _spectrum_kernel(fct_ref, fst_ref, qp_ref, kp_ref, pr_ref, pi_ref):
    fc = fct_ref[...]
    fs = fst_ref[...]
    qp = qp_ref[0]
    kp = kp_ref[0]
    qr = jnp.dot(fc, qp, preferred_element_type=jnp.float32, precision=_PREC)
    qi = jnp.dot(fs, qp, preferred_element_type=jnp.float32, precision=_PREC)
    kr = jnp.dot(fc, kp, preferred_element_type=jnp.float32, precision=_PREC)
    ki = jnp.dot(fs, kp, preferred_element_type=jnp.float32, precision=_PREC)
    pr_ref[0] = qr * kr + qi * ki
    pi_ref[0] = qi * kr - qr * ki


def _spectrum(fct, fst, qp, kp, bk=384, bc=512):
    return pl.pallas_call(
        _spectrum_kernel,
        grid=(_B, _NFP // bk, _DM // bc),
        in_specs=[pl.BlockSpec((bk, _L), lambda b, j, c: (j, 0)),
                  pl.BlockSpec((bk, _L), lambda b, j, c: (j, 0)),
                  pl.BlockSpec((1, _L, bc), lambda b, j, c: (b, 0, c)),
                  pl.BlockSpec((1, _L, bc), lambda b, j, c: (b, 0, c))],
        out_specs=[pl.BlockSpec((1, bk, bc), lambda b, j, c: (b, j, c)),
                   pl.BlockSpec((1, bk, bc), lambda b, j, c: (b, j, c))],
        out_shape=[jax.ShapeDtypeStruct((_B, _NFP, _DM), jnp.float32),
                   jax.ShapeDtypeStruct((_B, _NFP, _DM), jnp.float32)],
    )(fct, fst, qp, kp)


def _corr_kernel(act_ref, ast_ref, pr_ref, pi_ref, o_ref):
    o_ref[0] = (jnp.dot(act_ref[...], pr_ref[0],
                        preferred_element_type=jnp.float32, precision=_PREC) +
                jnp.dot(ast_ref[...], pi_ref[0],
                        preferred_element_type=jnp.float32, precision=_PREC))


def _corr(act, ast, pr, pi, bl=512, bc=512):
    return pl.pallas_call(
        _corr_kernel,
        grid=(_B, _L // bl, _DM // bc),
        in_specs=[pl.BlockSpec((bl, _NFP), lambda b, i, c: (i, 0)),
                  pl.BlockSpec((bl, _NFP), lambda b, i, c: (i, 0)),
                  pl.BlockSpec((1, _NFP, bc), lambda b, i, c: (b, 0, c)),
                  pl.BlockSpec((1, _NFP, bc), lambda b, i, c: (b, 0, c))],
        out_specs=pl.BlockSpec((1, bl, bc), lambda b, i, c: (b, i, c)),
        out_shape=jax.ShapeDtypeStruct((_B, _L, _DM), jnp.float32),
    )(act, ast, pr, pi)


def _topk_kernel(corr_ref, mt_ref, *, bc):
    j = pl.program_id(1)
    corr = corr_ref[0]
    iota = lax.broadcasted_iota(jnp.int32, corr.shape, 0)
    neg = jnp.float32(-3.0e38)
    vals = []
    taus = []
    c = corr
    for _ in range(_K):
        m = jnp.max(c, axis=0, keepdims=True)
        idx = jnp.min(jnp.where(c == m, iota, _L), axis=0, keepdims=True)
        vals.append(m)
        taus.append(idx)
        c = jnp.where(iota == idx, neg, c)
    v = jnp.concatenate(vals, axis=0)
    tau = jnp.concatenate(taus, axis=0)
    v = v - jnp.max(v, axis=0, keepdims=True)
    e = jnp.exp(v)
    w = e / jnp.sum(e, axis=0, keepdims=True)
    tch = j * bc + lax.broadcasted_iota(jnp.int32, (1, bc), 1)
    head_base = (tch // _DH) * _DH
    tloc = tch % _DH
    iota_s = lax.broadcasted_iota(jnp.int32, (_DM, bc), 0)
    mt = jnp.zeros((_DM, bc), jnp.float32)
    for i in range(_K):
        src = head_base + lax.rem(tloc - tau[i:i + 1, :] + _L * _DH, _DH)
        mt = mt + jnp.where(iota_s == src, w[i:i + 1, :], 0.0)
    mt_ref[0] = mt


def _topk(corr, bc=256):
    return pl.pallas_call(
        functools.partial(_topk_kernel, bc=bc),
        grid=(_B, _DM // bc),
        in_specs=[pl.BlockSpec((1, _L, bc), lambda b, j: (b, 0, j))],
        out_specs=pl.BlockSpec((1, _DM, bc), lambda b, j: (b, 0, j)),
        out_shape=jax.ShapeDtypeStruct((_B, _DM, _DM), jnp.float32),
    )(corr)


def _mix_kernel(vp_ref, mt_ref, o_ref):
    o_ref[0] = jnp.dot(vp_ref[0], mt_ref[0],
                       preferred_element_type=jnp.float32, precision=_PREC)


def _mix(vp, mt, bl=512):
    return pl.pallas_call(
        _mix_kernel,
        grid=(_B, _L // bl),
        in_specs=[pl.BlockSpec((1, bl, _DM), lambda b, i: (b, i, 0)),
                  pl.BlockSpec((1, _DM, _DM), lambda b, i: (b, 0, 0))],
        out_specs=pl.BlockSpec((1, bl, _DM), lambda b, i: (b, i, 0)),
        out_shape=jax.ShapeDtypeStruct((_B, _L, _DM), jnp.float32),
    )(vp, mt)


def kernel(queries, keys, values, wq, wk, wv, wo):
    fct, fst, act, ast = _dft_consts()
    q2 = queries.reshape(_B * _L, _DM)
    k2 = keys.reshape(_B * _L, _DM)
    v2 = values.reshape(_B * _L, _DM)
    qp = _matmul(q2, wq).reshape(_B, _L, _DM)
    kp = _matmul(k2, wk).reshape(_B, _L, _DM)
    vp = _matmul(v2, wv).reshape(_B, _L, _DM)
    pr, pi = _spectrum(fct, fst, qp, kp)
    corr = _corr(act, ast, pr, pi)
    mt = _topk(corr)
    oc = _mix(vp, mt)
    r = oc.reshape(_B, 2, _DM, _H, _DH).transpose(0, 4, 3, 1, 2)
    r = r.reshape(_B * _L, _DM)
    out = _matmul(r, wo)
    return out.reshape(_B, _L, _DM)


# speedup vs baseline: 4.4925x; 1.0618x over previous
import functools
import math

import numpy as np
import jax
import jax.numpy as jnp
from jax import lax
from jax.experimental import pallas as pl

_B = 2
_L = 2048
_DM = 1024
_H = 16
_DH = 64
_K = int(math.floor(math.log(_L)))
_NF = 1024

_PREC_DFT = lax.Precision.HIGHEST
_PREC_PROJ = lax.Precision.DEFAULT
_PREC_MIX = lax.Precision.HIGHEST


def _dft_consts():
    l = np.arange(_L, dtype=np.float64)
    k = np.arange(_NF, dtype=np.float64)
    ang = 2.0 * np.pi * np.outer(k, l) / _L
    fct = np.cos(ang)
    fst = -np.sin(ang)
    ck = np.where(np.arange(_NF) == 0, 1.0, 2.0) / _L
    act = (np.cos(ang) * ck[:, None]).T
    ast = (-np.sin(ang) * ck[:, None]).T
    fnq = np.zeros((8, _L))
    fnq[0] = np.where(l.astype(np.int64) % 2 == 0, 1.0, -1.0)
    return (jnp.asarray(fct, jnp.float32), jnp.asarray(fst, jnp.float32),
            jnp.asarray(act, jnp.float32), jnp.asarray(ast, jnp.float32),
            jnp.asarray(fnq, jnp.float32))


def _mm_kernel(x_ref, w_ref, o_ref):
    o_ref[...] = jnp.dot(x_ref[...], w_ref[...],
                         preferred_element_type=jnp.float32,
                         precision=_PREC_PROJ)


def _matmul(x, w, bm=512):
    m, kk = x.shape
    n = w.shape[1]
    return pl.pallas_call(
        _mm_kernel,
        grid=(m // bm,),
        in_specs=[pl.BlockSpec((bm, kk), lambda i: (i, 0)),
                  pl.BlockSpec((kk, n), lambda i: (0, 0))],
        out_specs=pl.BlockSpec((bm, n), lambda i: (i, 0)),
        out_shape=jax.ShapeDtypeStruct((m, n), jnp.float32),
    )(x, w)


def _spectrum_kernel(fct_ref, fst_ref, fnq_ref, qp_ref, kp_ref,
                     pr_ref, pi_ref, pn_ref):
    fc = fct_ref[...]
    fs = fst_ref[...]
    qp = qp_ref[0]
    kp = kp_ref[0]
    qr = jnp.dot(fc, qp, preferred_element_type=jnp.float32,
                 precision=_PREC_DFT)
    qi = jnp.dot(fs, qp, preferred_element_type=jnp.float32,
                 precision=_PREC_DFT)
    kr = jnp.dot(fc, kp, preferred_element_type=jnp.float32,
                 precision=_PREC_DFT)
    ki = jnp.dot(fs, kp, preferred_element_type=jnp.float32,
                 precision=_PREC_DFT)
    pr_ref[0] = qr * kr + qi * ki
    pi_ref[0] = qi * kr - qr * ki

    @pl.when(pl.program_id(1) == 0)
    def _():
        fn = fnq_ref[...]
        qn = jnp.dot(fn, qp, preferred_element_type=jnp.float32,
                     precision=_PREC_DFT)
        kn = jnp.dot(fn, kp, preferred_element_type=jnp.float32,
                     precision=_PREC_DFT)
        pn_ref[0] = qn * kn * jnp.float32(1.0 / _L)


def _spectrum(fct, fst, fnq, qp, kp, bk=256, bc=512):
    return pl.pallas_call(
        _spectrum_kernel,
        grid=(_B, _NF // bk, _DM // bc),
        in_specs=[pl.BlockSpec((bk, _L), lambda b, j, c: (j, 0)),
                  pl.BlockSpec((bk, _L), lambda b, j, c: (j, 0)),
                  pl.BlockSpec((8, _L), lambda b, j, c: (0, 0)),
                  pl.BlockSpec((1, _L, bc), lambda b, j, c: (b, 0, c)),
                  pl.BlockSpec((1, _L, bc), lambda b, j, c: (b, 0, c))],
        out_specs=[pl.BlockSpec((1, bk, bc), lambda b, j, c: (b, j, c)),
                   pl.BlockSpec((1, bk, bc), lambda b, j, c: (b, j, c)),
                   pl.BlockSpec((1, 8, bc), lambda b, j, c: (b, 0, c))],
        out_shape=[jax.ShapeDtypeStruct((_B, _NF, _DM), jnp.float32),
                   jax.ShapeDtypeStruct((_B, _NF, _DM), jnp.float32),
                   jax.ShapeDtypeStruct((_B, 8, _DM), jnp.float32)],
    )(fct, fst, fnq, qp, kp)


def _corr_kernel(act_ref, ast_ref, pr_ref, pi_ref, pn_ref, o_ref, *, bl):
    i = pl.program_id(1)
    c = (jnp.dot(act_ref[...], pr_ref[0],
                 preferred_element_type=jnp.float32, precision=_PREC_DFT) +
         jnp.dot(ast_ref[...], pi_ref[0],
                 preferred_element_type=jnp.float32, precision=_PREC_DFT))
    n_iota = i * bl + lax.broadcasted_iota(jnp.int32, (bl, 1), 0)
    sign = jnp.where(n_iota % 2 == 0, jnp.float32(1.0), jnp.float32(-1.0))
    o_ref[0] = c + sign * pn_ref[0, 0:1, :]


def _corr(act, ast, pr, pi, pn, bl=512, bc=512):
    return pl.pallas_call(
        functools.partial(_corr_kernel, bl=bl),
        grid=(_B, _L // bl, _DM // bc),
        in_specs=[pl.BlockSpec((bl, _NF), lambda b, i, c: (i, 0)),
                  pl.BlockSpec((bl, _NF), lambda b, i, c: (i, 0)),
                  pl.BlockSpec((1, _NF, bc), lambda b, i, c: (b, 0, c)),
                  pl.BlockSpec((1, _NF, bc), lambda b, i, c: (b, 0, c)),
                  pl.BlockSpec((1, 8, bc), lambda b, i, c: (b, 0, c))],
        out_specs=pl.BlockSpec((1, bl, bc), lambda b, i, c: (b, i, c)),
        out_shape=jax.ShapeDtypeStruct((_B, _L, _DM), jnp.float32),
    )(act, ast, pr, pi, pn)


def _topk_kernel(corr_ref, mt_ref, *, bc):
    j = pl.program_id(1)
    corr = corr_ref[0]
    iota = lax.broadcasted_iota(jnp.int32, corr.shape, 0)
    neg = jnp.float32(-3.0e38)
    vals = []
    taus = []
    c = corr
    for _ in range(_K):
        m = jnp.max(c, axis=0, keepdims=True)
        idx = jnp.min(jnp.where(c == m, iota, _L), axis=0, keepdims=True)
        vals.append(m)
        taus.append(idx)
        c = jnp.where(iota == idx, neg, c)
    v = jnp.concatenate(vals, axis=0)
    tau = jnp.concatenate(taus, axis=0)
    v = v - jnp.max(v, axis=0, keepdims=True)
    e = jnp.exp(v)
    w = e / jnp.sum(e, axis=0, keepdims=True)
    tch = j * bc + lax.broadcasted_iota(jnp.int32, (1, bc), 1)
    head_base = (tch // _DH) * _DH
    tloc = tch % _DH
    iota_s = lax.broadcasted_iota(jnp.int32, (_DM, bc), 0)
    mt = jnp.zeros((_DM, bc), jnp.float32)
    for i in range(_K):
        src = head_base + lax.rem(tloc - tau[i:i + 1, :] + _L * _DH, _DH)
        mt = mt + jnp.where(iota_s == src, w[i:i + 1, :], 0.0)
    mt_ref[0] = mt


def _topk(corr, bc=256):
    return pl.pallas_call(
        functools.partial(_topk_kernel, bc=bc),
        grid=(_B, _DM // bc),
        in_specs=[pl.BlockSpec((1, _L, bc), lambda b, j: (b, 0, j))],
        out_specs=pl.BlockSpec((1, _DM, bc), lambda b, j: (b, 0, j)),
        out_shape=jax.ShapeDtypeStruct((_B, _DM, _DM), jnp.float32),
    )(corr)


def _mix_kernel(vp_ref, mt_ref, o_ref):
    o_ref[0] = jnp.dot(vp_ref[0], mt_ref[0],
                       preferred_element_type=jnp.float32,
                       precision=_PREC_MIX)


def _mix(vp, mt, bl=512):
    return pl.pallas_call(
        _mix_kernel,
        grid=(_B, _L // bl),
        in_specs=[pl.BlockSpec((1, bl, _DM), lambda b, i: (b, i, 0)),
                  pl.BlockSpec((1, _DM, _DM), lambda b, i: (b, 0, 0))],
        out_specs=pl.BlockSpec((1, bl, _DM), lambda b, i: (b, i, 0)),
        out_shape=jax.ShapeDtypeStruct((_B, _L, _DM), jnp.float32),
    )(vp, mt)


def kernel(queries, keys, values, wq, wk, wv, wo):
    fct, fst, act, ast, fnq = _dft_consts()
    q2 = queries.reshape(_B * _L, _DM)
    k2 = keys.reshape(_B * _L, _DM)
    v2 = values.reshape(_B * _L, _DM)
    qp = _matmul(q2, wq).reshape(_B, _L, _DM)
    kp = _matmul(k2, wk).reshape(_B, _L, _DM)
    vp = _matmul(v2, wv).reshape(_B, _L, _DM)
    pr, pi, pn = _spectrum(fct, fst, fnq, qp, kp)
    corr = _corr(act, ast, pr, pi, pn)
    mt = _topk(corr)
    oc = _mix(vp, mt)
    r = oc.reshape(_B, 2, _DM, _H, _DH).transpose(0, 4, 3, 1, 2)
    r = r.reshape(_B * _L, _DM)
    out = _matmul(r, wo)
    return out.reshape(_B, _L, _DM)
